# speedup vs baseline: 2.5446x; 2.5446x over previous
import functools

import jax
import jax.numpy as jnp
from jax import lax
from jax.experimental import pallas as pl
from jax.experimental.pallas import tpu as pltpu
from jax.experimental.pallas import tpu_sc as plsc

HID = 128
CHUNK = 128


@functools.lru_cache(maxsize=None)
def _make_gather(n_rows: int, hid: int):
    info = plsc.get_sparse_core_info()
    nc, ns = info.num_cores, info.num_subcores
    nw = nc * ns
    assert n_rows % (nw * CHUNK) == 0
    b_per_w = n_rows // nw
    n_chunks = b_per_w // CHUNK
    mesh = plsc.VectorSubcoreMesh(core_axis_name="c", subcore_axis_name="s")

    @functools.partial(
        pl.kernel,
        mesh=mesh,
        out_type=jax.ShapeDtypeStruct((n_rows, hid), jnp.float32),
        scratch_types=[
            pltpu.VMEM((b_per_w,), jnp.int32),
            pltpu.VMEM((2, CHUNK, hid), jnp.float32),
            pltpu.SemaphoreType.DMA,
            pltpu.SemaphoreType.DMA,
        ],
    )
    def gather(table_hbm, idx_hbm, out_hbm, idx_v, rows_v, sem0, sem1):
        sems = (sem0, sem1)
        wid = lax.axis_index("s") * nc + lax.axis_index("c")
        base = wid * b_per_w
        pltpu.sync_copy(idx_hbm.at[pl.ds(base, b_per_w)], idx_v)
        handles = [None, None]
        handles[0] = pltpu.async_copy(
            table_hbm.at[idx_v.at[pl.ds(0, CHUNK)]], rows_v.at[0], sems[0]
        )
        for c in range(n_chunks):
            buf = c % 2
            nxt = (c + 1) % 2
            if c + 1 < n_chunks:
                handles[nxt] = pltpu.async_copy(
                    table_hbm.at[idx_v.at[pl.ds((c + 1) * CHUNK, CHUNK)]],
                    rows_v.at[nxt],
                    sems[nxt],
                )
            handles[buf].wait()
            pltpu.sync_copy(
                rows_v.at[buf], out_hbm.at[pl.ds(base + c * CHUNK, CHUNK)]
            )

    return gather


def _matmul_block(x_ref, w_ref, b_ref, o_ref):
    o_ref[...] = (
        jnp.dot(x_ref[...], w_ref[...], preferred_element_type=jnp.float32)
        + b_ref[...]
    )


def _project(x, w, bias):
    n, hid = x.shape
    blk = 2048
    assert n % blk == 0
    return pl.pallas_call(
        _matmul_block,
        grid=(n // blk,),
        in_specs=[
            pl.BlockSpec((blk, hid), lambda i: (i, 0)),
            pl.BlockSpec((hid, hid), lambda i: (0, 0)),
            pl.BlockSpec((1, hid), lambda i: (0, 0)),
        ],
        out_specs=pl.BlockSpec((blk, hid), lambda i: (i, 0)),
        out_shape=jax.ShapeDtypeStruct((n, hid), jnp.float32),
    )(x, w, bias.reshape(1, hid))


def kernel(input_ids, table, W, b):
    bsz, num_terms, _ = input_ids.shape
    idx = input_ids[:, :, 0].reshape(-1)
    gathered = _make_gather(idx.shape[0], HID)(table, idx)
    out = _project(gathered, W, b)
    return out.reshape(bsz, num_terms, HID)


# speedup vs baseline: 4.6172x; 1.8145x over previous
import functools

import jax
import jax.numpy as jnp
from jax import lax
from jax.experimental import pallas as pl
from jax.experimental.pallas import tpu as pltpu
from jax.experimental.pallas import tpu_sc as plsc

HID = 128
CHUNK = 128


@functools.lru_cache(maxsize=None)
def _make_gather(n_rows: int, hid: int):
    info = plsc.get_sparse_core_info()
    nc, ns = info.num_cores, info.num_subcores
    nw = nc * ns
    assert n_rows % (nw * CHUNK) == 0
    b_per_w = n_rows // nw
    n_chunks = b_per_w // CHUNK
    mesh = plsc.VectorSubcoreMesh(core_axis_name="c", subcore_axis_name="s")

    @functools.partial(
        pl.kernel,
        mesh=mesh,
        out_type=jax.ShapeDtypeStruct((n_rows, hid), jnp.float32),
        scratch_types=[
            pltpu.VMEM((b_per_w,), jnp.int32),
            pltpu.VMEM((2, CHUNK, hid), jnp.float32),
            pltpu.SemaphoreType.DMA,
            pltpu.SemaphoreType.DMA,
        ],
    )
    def gather(table_hbm, idx_hbm, out_hbm, idx_v, rows_v, sem0, sem1):
        sems = (sem0, sem1)
        wid = lax.axis_index("s") * nc + lax.axis_index("c")
        base = wid * b_per_w
        pltpu.sync_copy(idx_hbm.at[pl.ds(base, b_per_w)], idx_v)
        handles = [None, None]
        handles[0] = pltpu.async_copy(
            table_hbm.at[idx_v.at[pl.ds(0, CHUNK)]], rows_v.at[0], sems[0]
        )
        for c in range(n_chunks):
            buf = c % 2
            nxt = (c + 1) % 2
            if c + 1 < n_chunks:
                handles[nxt] = pltpu.async_copy(
                    table_hbm.at[idx_v.at[pl.ds((c + 1) * CHUNK, CHUNK)]],
                    rows_v.at[nxt],
                    sems[nxt],
                )
            handles[buf].wait()
            pltpu.sync_copy(
                rows_v.at[buf], out_hbm.at[pl.ds(base + c * CHUNK, CHUNK)]
            )

    return gather


def _matmul_block(x_ref, w_ref, b_ref, o_ref):
    o_ref[...] = (
        jnp.dot(x_ref[...], w_ref[...], preferred_element_type=jnp.float32)
        + b_ref[...]
    )


def _project(x, w, bias):
    n, hid = x.shape
    blk = 2048
    assert n % blk == 0
    return pl.pallas_call(
        _matmul_block,
        grid=(n // blk,),
        in_specs=[
            pl.BlockSpec((blk, hid), lambda i: (i, 0)),
            pl.BlockSpec((hid, hid), lambda i: (0, 0)),
            pl.BlockSpec((1, hid), lambda i: (0, 0)),
        ],
        out_specs=pl.BlockSpec((blk, hid), lambda i: (i, 0)),
        out_shape=jax.ShapeDtypeStruct((n, hid), jnp.float32),
    )(x, w, bias.reshape(1, hid))


def kernel(input_ids, table, W, b):
    bsz, num_terms, _ = input_ids.shape
    idx = input_ids[:, :, 0].T.reshape(-1)
    gathered = _make_gather(idx.shape[0], HID)(table, idx)
    out = _project(gathered, W, b)
    return out.reshape(num_terms, bsz, HID).transpose(1, 0, 2)


# speedup vs baseline: 4.8836x; 1.0577x over previous
import functools

import jax
import jax.numpy as jnp
from jax import lax
from jax.experimental import pallas as pl
from jax.experimental.pallas import tpu as pltpu
from jax.experimental.pallas import tpu_sc as plsc

HID = 128
CHUNK = 128


@functools.lru_cache(maxsize=None)
def _make_gather(n_rows: int, hid: int):
    info = plsc.get_sparse_core_info()
    nc, ns = info.num_cores, info.num_subcores
    nw = nc * ns
    assert n_rows % (nw * CHUNK) == 0
    b_per_w = n_rows // nw
    n_chunks = b_per_w // CHUNK
    mesh = plsc.VectorSubcoreMesh(core_axis_name="c", subcore_axis_name="s")

    @functools.partial(
        pl.kernel,
        mesh=mesh,
        out_type=jax.ShapeDtypeStruct((n_rows, hid), jnp.float32),
        scratch_types=[
            pltpu.VMEM((b_per_w,), jnp.int32),
            pltpu.VMEM((2, CHUNK, hid), jnp.float32),
            pltpu.SemaphoreType.DMA,
            pltpu.SemaphoreType.DMA,
        ],
    )
    def gather(table_hbm, idx_hbm, out_hbm, idx_v, rows_v, sem0, sem1):
        sems = (sem0, sem1)
        wid = lax.axis_index("s") * nc + lax.axis_index("c")
        base = wid * b_per_w
        pltpu.sync_copy(idx_hbm.at[pl.ds(base, b_per_w)], idx_v)
        handles = [None, None]
        handles[0] = pltpu.async_copy(
            table_hbm.at[idx_v.at[pl.ds(0, CHUNK)]], rows_v.at[0], sems[0]
        )
        for c in range(n_chunks):
            buf = c % 2
            nxt = (c + 1) % 2
            if c + 1 < n_chunks:
                handles[nxt] = pltpu.async_copy(
                    table_hbm.at[idx_v.at[pl.ds((c + 1) * CHUNK, CHUNK)]],
                    rows_v.at[nxt],
                    sems[nxt],
                )
            handles[buf].wait()
            pltpu.sync_copy(
                rows_v.at[buf], out_hbm.at[pl.ds(base + c * CHUNK, CHUNK)]
            )

    return gather


def _matmul_block(x_ref, w_ref, b_ref, o_ref):
    o_ref[...] = (
        jnp.dot(x_ref[...], w_ref[...], preferred_element_type=jnp.float32)
        + b_ref[...]
    )


def _matmul_block_into(buf_ref, x_ref, w_ref, b_ref, o_ref):
    del buf_ref
    o_ref[...] = (
        jnp.dot(x_ref[...], w_ref[...], preferred_element_type=jnp.float32)
        + b_ref[...]
    )


BLK = 2048
PIPE = 4


def _project_chunk(buf, x, w, bias2d, g, n_total):
    n, hid = x.shape
    nblk = n // BLK
    off = g * nblk
    x_spec = pl.BlockSpec((BLK, hid), lambda i: (i, 0))
    w_spec = pl.BlockSpec((hid, hid), lambda i: (0, 0))
    b_spec = pl.BlockSpec((1, hid), lambda i: (0, 0))
    out_spec = pl.BlockSpec((BLK, hid), lambda i, o=off: (i + o, 0))
    out_shape = jax.ShapeDtypeStruct((n_total, hid), jnp.float32)
    if buf is None:
        return pl.pallas_call(
            _matmul_block,
            grid=(nblk,),
            in_specs=[x_spec, w_spec, b_spec],
            out_specs=out_spec,
            out_shape=out_shape,
        )(x, w, bias2d)
    return pl.pallas_call(
        _matmul_block_into,
        grid=(nblk,),
        in_specs=[
            pl.BlockSpec(memory_space=pl.ANY),
            x_spec,
            w_spec,
            b_spec,
        ],
        out_specs=out_spec,
        out_shape=out_shape,
        input_output_aliases={0: 0},
    )(buf, x, w, bias2d)


def kernel(input_ids, table, W, b):
    bsz, num_terms, _ = input_ids.shape
    idx = input_ids[:, :, 0].T.reshape(-1)
    n = idx.shape[0]
    rows_pc = n // PIPE
    gather = _make_gather(rows_pc, HID)
    chunks = [
        gather(table, lax.slice(idx, (g * rows_pc,), ((g + 1) * rows_pc,)))
        for g in range(PIPE)
    ]
    bias2d = b.reshape(1, HID)
    out = None
    for g, ch in enumerate(chunks):
        out = _project_chunk(out, ch, W, bias2d, g, n)
    return out.reshape(num_terms, bsz, HID).transpose(1, 0, 2)
